# baseline (speedup 1.0000x reference)
import jax
import jax.numpy as jnp
from jax import lax
from jax.experimental import pallas as pl
from jax.experimental.pallas import tpu as pltpu
from jax.experimental.pallas import tpu_sc as plsc

B = 16384
D = 32
NT = 100000
_NC, _NS = 2, 16
_NW = _NC * _NS
_BPW = B // _NW
_CH = 128
_NCH = _BPW // _CH
_IDX_ROWS = B // _CH


def _mk_gather(nsets):
    def body(*args):
        idx_h = args[0:nsets]
        tbl = args[nsets]
        outs = args[nsets + 1:2 * nsets + 1]
        scratch = args[2 * nsets + 1:]
        idxv = scratch[0:nsets]
        bufs = scratch[nsets:3 * nsets]
        sem = scratch[-1]
        wid = lax.axis_index("s") * _NC + lax.axis_index("c")
        rbase = wid * _BPW
        ibase = wid * _NCH
        for k in range(nsets):
            pltpu.sync_copy(idx_h[k].at[pl.ds(ibase, _NCH)], idxv[k])

        def fire(c):
            return [pltpu.async_copy(tbl.at[idxv[k].at[c]],
                                     bufs[2 * k + c % 2], sem)
                    for k in range(nsets)]

        def copy_out(c):
            for k in range(nsets):
                pltpu.sync_copy(bufs[2 * k + c % 2],
                                outs[k].at[pl.ds(rbase + c * _CH, _CH)])

        prev = fire(0)
        for c in range(1, _NCH):
            cur = fire(c)
            for d in prev:
                d.wait()
            copy_out(c - 1)
            prev = cur
        for d in prev:
            d.wait()
        copy_out(_NCH - 1)

    return pl.kernel(
        body,
        out_type=[jax.ShapeDtypeStruct((B, D), jnp.float32)] * nsets,
        mesh=plsc.VectorSubcoreMesh(core_axis_name="c", subcore_axis_name="s"),
        compiler_params=pltpu.CompilerParams(use_tc_tiling_on_sc=False),
        scratch_types=(
            [pltpu.VMEM((_NCH, _CH), jnp.int32)] * nsets
            + [pltpu.VMEM((_CH, D), jnp.float32)] * (2 * nsets)
            + [pltpu.SemaphoreType.DMA]
        ),
    )


_g1 = _mk_gather(1)
_g2 = _mk_gather(2)

_BT = 2048
_GRID = (B // 4) // _BT
_PK = B // 4


def _tc_body(ar_ref, tr_ref,
             aue, tue, aie, tie, waa, wat, wba, wbt,
             w1au, b1au, w2au, b2au,
             w1tu, b1tu, w2tu, b2tu,
             w1ai, b1ai, w2ai, b2ai,
             w1ti, b1ti, w2ti, b2ti,
             sel, la_ref, lt_ref,
             w1s0, w1s1, w1s2, w1s3, w2s0, w2s1, w2s2, w2s3):
    i = pl.program_id(0)

    @pl.when(i == 0)
    def _():
        for ws, w, d_in, d_out in (
                (w1s0, w1au, D, 2 * D), (w1s1, w1tu, D, 2 * D),
                (w1s2, w1ai, D, 2 * D), (w1s3, w1ti, D, 2 * D),
                (w2s0, w2au, 2 * D, D), (w2s1, w2tu, 2 * D, D),
                (w2s2, w2ai, 2 * D, D), (w2s3, w2ti, 2 * D, D)):
            ws[...] = jnp.zeros((4 * d_in, 4 * d_out), jnp.float32)
            for k in range(4):
                ws[k * d_in:(k + 1) * d_in, k * d_out:(k + 1) * d_out] = w[...]
    a_e = aue[...].astype(jnp.float32)
    t_e = tue[...].astype(jnp.float32)
    x_au = waa[...].astype(jnp.float32) * a_e + \
        (1.0 - wat[...].astype(jnp.float32)) * t_e
    x_tu = wba[...].astype(jnp.float32) * a_e + \
        (1.0 - wbt[...].astype(jnp.float32)) * t_e
    x_ai = aie[...].astype(jnp.float32)
    x_ti = tie[...].astype(jnp.float32)

    def mlp(x, w1, b1, w2, b2):
        b1t = jnp.concatenate([b1[...]] * 4, axis=1)
        b2t = jnp.concatenate([b2[...]] * 4, axis=1)
        h = jnp.maximum(
            jnp.dot(x, w1[...], preferred_element_type=jnp.float32)
            + b1t, 0.0)
        return jnp.maximum(
            jnp.dot(h, w2[...], preferred_element_type=jnp.float32)
            + b2t, 0.0)

    y_au = mlp(x_au, w1s0, b1au, w2s0, b2au)
    y_tu = mlp(x_tu, w1s1, b1tu, w2s1, b2tu)
    y_ai = mlp(x_ai, w1s2, b1ai, w2s2, b2ai)
    y_ti = mlp(x_ti, w1s3, b1ti, w2s3, b2ti)

    s_a = jnp.maximum(
        jnp.dot(y_au * y_ai, sel[...], preferred_element_type=jnp.float32),
        1e-6)
    s_t = jnp.maximum(
        jnp.dot(y_tu * y_ti, sel[...], preferred_element_type=jnp.float32),
        1e-6)
    da = s_a - ar_ref[...].astype(jnp.float32)
    dt = s_t - tr_ref[...].astype(jnp.float32)
    pa = jnp.sum(da * da) * (1.0 / B)
    pt = jnp.sum(dt * dt) * (1.0 / B)

    @pl.when(i == 0)
    def _():
        la_ref[0, 0] = 0.0
        lt_ref[0, 0] = 0.0

    la_ref[0, 0] += pa
    lt_ref[0, 0] += pt


def _wspec():
    return pl.BlockSpec((D, 2 * D), lambda i: (0, 0))


def _bspec():
    return pl.BlockSpec((1, 2 * D), lambda i: (0, 0))


def _w2spec():
    return pl.BlockSpec((2 * D, D), lambda i: (0, 0))


def _b2spec():
    return pl.BlockSpec((1, D), lambda i: (0, 0))


_tc_dense = pl.pallas_call(
    _tc_body,
    grid=(_GRID,),
    in_specs=[
        pl.BlockSpec((_BT, 4), lambda i: (i, 0)),
        pl.BlockSpec((_BT, 4), lambda i: (i, 0)),
    ] + [pl.BlockSpec((_BT, 4 * D), lambda i: (i, 0))] * 8 + [
        _wspec(), _bspec(), _w2spec(), _b2spec(),
        _wspec(), _bspec(), _w2spec(), _b2spec(),
        _wspec(), _bspec(), _w2spec(), _b2spec(),
        _wspec(), _bspec(), _w2spec(), _b2spec(),
        pl.BlockSpec((4 * D, 4), lambda i: (0, 0)),
    ],
    out_specs=[
        pl.BlockSpec(memory_space=pltpu.SMEM),
        pl.BlockSpec(memory_space=pltpu.SMEM),
    ],
    out_shape=[jax.ShapeDtypeStruct((1, 1), jnp.float32)] * 2,
    scratch_shapes=(
        [pltpu.VMEM((4 * D, 8 * D), jnp.float32)] * 4
        + [pltpu.VMEM((8 * D, 4 * D), jnp.float32)] * 4
    ),
)


def kernel(ausers, aitems, aratings, tusers, titems, tratings, params):
    p = params
    au2 = ausers.reshape(_IDX_ROWS, _CH)
    tu2 = tusers.reshape(_IDX_ROWS, _CH)
    ai2 = aitems.reshape(_IDX_ROWS, _CH)
    ti2 = titems.reshape(_IDX_ROWS, _CH)
    (aue_g,) = _g1(au2, p["a_emb_user"])
    (tue_g,) = _g1(tu2, p["t_emb_user"])
    (aie_g,) = _g1(ai2, p["a_emb_item"])
    (tie_g,) = _g1(ti2, p["t_emb_item"])
    waa_g, wat_g = _g2(au2, tu2, p["W_a"])
    wba_g, wbt_g = _g2(au2, tu2, p["W_b"])
    gathered = [aue_g, tue_g, aie_g, tie_g, waa_g, wat_g, wba_g, wbt_g]
    packed = [g.reshape(_PK, 4 * D) for g in gathered]

    wargs = []
    for name in ("mlp_a_users", "mlp_t_users", "mlp_a_items", "mlp_t_items"):
        m = p[name]
        wargs += [
            m["W1"],
            m["b1"].reshape(1, 2 * D),
            m["W2"],
            m["b2"].reshape(1, D),
        ]
    sel = (jnp.arange(4 * D)[:, None] // D ==
           jnp.arange(4)[None, :]).astype(jnp.float32)

    ar2 = aratings.reshape(_PK, 4)
    tr2 = tratings.reshape(_PK, 4)
    la, lt = _tc_dense(ar2, tr2, *packed, *wargs, sel)
    return (la[0, 0], lt[0, 0])


# speedup vs baseline: 1.0051x; 1.0051x over previous
import jax
import jax.numpy as jnp
from jax import lax
from jax.experimental import pallas as pl
from jax.experimental.pallas import tpu as pltpu
from jax.experimental.pallas import tpu_sc as plsc

B = 16384
D = 32
NT = 100000
_NC, _NS = 2, 16
_NW = _NC * _NS
_BPW = B // _NW
_CH = 128
_NCH = _BPW // _CH
_IDX_ROWS = B // _CH


def _mk_gather(nsets):
    def body(*args):
        idx_h = args[0:nsets]
        tbl = args[nsets]
        outs = args[nsets + 1:2 * nsets + 1]
        scratch = args[2 * nsets + 1:]
        idxv = scratch[0:nsets]
        bufs = scratch[nsets:3 * nsets]
        sem = scratch[-1]
        wid = lax.axis_index("s") * _NC + lax.axis_index("c")
        rbase = wid * _BPW
        ibase = wid * _NCH
        for k in range(nsets):
            pltpu.sync_copy(idx_h[k].at[pl.ds(ibase, _NCH)], idxv[k])

        def fire(c):
            return [pltpu.async_copy(tbl.at[idxv[k].at[c]],
                                     bufs[2 * k + c % 2], sem)
                    for k in range(nsets)]

        def copy_out(c):
            for k in range(nsets):
                pltpu.sync_copy(bufs[2 * k + c % 2],
                                outs[k].at[pl.ds(rbase + c * _CH, _CH)])

        prev = fire(0)
        for c in range(1, _NCH):
            cur = fire(c)
            for d in prev:
                d.wait()
            copy_out(c - 1)
            prev = cur
        for d in prev:
            d.wait()
        copy_out(_NCH - 1)

    return pl.kernel(
        body,
        out_type=[jax.ShapeDtypeStruct((B, D), jnp.float32)] * nsets,
        mesh=plsc.VectorSubcoreMesh(core_axis_name="c", subcore_axis_name="s"),
        compiler_params=pltpu.CompilerParams(use_tc_tiling_on_sc=False),
        scratch_types=(
            [pltpu.VMEM((_NCH, _CH), jnp.int32)] * nsets
            + [pltpu.VMEM((_CH, D), jnp.float32)] * (2 * nsets)
            + [pltpu.SemaphoreType.DMA]
        ),
    )


_g1 = _mk_gather(1)
_g2 = _mk_gather(2)

_BT = 1024
_GRID = (B // 4) // _BT
_PK = B // 4


def _tc_body(ar_ref, tr_ref,
             aue, tue, aie, tie, waa, wat, wba, wbt,
             w1au, b1au, w2au, b2au,
             w1tu, b1tu, w2tu, b2tu,
             w1ai, b1ai, w2ai, b2ai,
             w1ti, b1ti, w2ti, b2ti,
             sel, la_ref, lt_ref,
             w1s0, w1s1, w1s2, w1s3, w2s0, w2s1, w2s2, w2s3):
    i = pl.program_id(0)

    @pl.when(i == 0)
    def _():
        for ws, w, d_in, d_out in (
                (w1s0, w1au, D, 2 * D), (w1s1, w1tu, D, 2 * D),
                (w1s2, w1ai, D, 2 * D), (w1s3, w1ti, D, 2 * D),
                (w2s0, w2au, 2 * D, D), (w2s1, w2tu, 2 * D, D),
                (w2s2, w2ai, 2 * D, D), (w2s3, w2ti, 2 * D, D)):
            ws[...] = jnp.zeros((4 * d_in, 4 * d_out), jnp.float32)
            for k in range(4):
                ws[k * d_in:(k + 1) * d_in, k * d_out:(k + 1) * d_out] = w[...]
    a_e = aue[...].astype(jnp.float32)
    t_e = tue[...].astype(jnp.float32)
    x_au = waa[...].astype(jnp.float32) * a_e + \
        (1.0 - wat[...].astype(jnp.float32)) * t_e
    x_tu = wba[...].astype(jnp.float32) * a_e + \
        (1.0 - wbt[...].astype(jnp.float32)) * t_e
    x_ai = aie[...].astype(jnp.float32)
    x_ti = tie[...].astype(jnp.float32)

    def mlp(x, w1, b1, w2, b2):
        b1t = jnp.concatenate([b1[...]] * 4, axis=1)
        b2t = jnp.concatenate([b2[...]] * 4, axis=1)
        h = jnp.maximum(
            jnp.dot(x, w1[...], preferred_element_type=jnp.float32)
            + b1t, 0.0)
        return jnp.maximum(
            jnp.dot(h, w2[...], preferred_element_type=jnp.float32)
            + b2t, 0.0)

    y_au = mlp(x_au, w1s0, b1au, w2s0, b2au)
    y_tu = mlp(x_tu, w1s1, b1tu, w2s1, b2tu)
    y_ai = mlp(x_ai, w1s2, b1ai, w2s2, b2ai)
    y_ti = mlp(x_ti, w1s3, b1ti, w2s3, b2ti)

    s_a = jnp.maximum(
        jnp.dot(y_au * y_ai, sel[...], preferred_element_type=jnp.float32),
        1e-6)
    s_t = jnp.maximum(
        jnp.dot(y_tu * y_ti, sel[...], preferred_element_type=jnp.float32),
        1e-6)
    da = s_a - ar_ref[...].astype(jnp.float32)
    dt = s_t - tr_ref[...].astype(jnp.float32)
    pa = jnp.sum(da * da) * (1.0 / B)
    pt = jnp.sum(dt * dt) * (1.0 / B)

    @pl.when(i == 0)
    def _():
        la_ref[0, 0] = 0.0
        lt_ref[0, 0] = 0.0

    la_ref[0, 0] += pa
    lt_ref[0, 0] += pt


def _wspec():
    return pl.BlockSpec((D, 2 * D), lambda i: (0, 0))


def _bspec():
    return pl.BlockSpec((1, 2 * D), lambda i: (0, 0))


def _w2spec():
    return pl.BlockSpec((2 * D, D), lambda i: (0, 0))


def _b2spec():
    return pl.BlockSpec((1, D), lambda i: (0, 0))


_tc_dense = pl.pallas_call(
    _tc_body,
    grid=(_GRID,),
    in_specs=[
        pl.BlockSpec((_BT, 4), lambda i: (i, 0)),
        pl.BlockSpec((_BT, 4), lambda i: (i, 0)),
    ] + [pl.BlockSpec((_BT, 4 * D), lambda i: (i, 0))] * 8 + [
        _wspec(), _bspec(), _w2spec(), _b2spec(),
        _wspec(), _bspec(), _w2spec(), _b2spec(),
        _wspec(), _bspec(), _w2spec(), _b2spec(),
        _wspec(), _bspec(), _w2spec(), _b2spec(),
        pl.BlockSpec((4 * D, 4), lambda i: (0, 0)),
    ],
    out_specs=[
        pl.BlockSpec(memory_space=pltpu.SMEM),
        pl.BlockSpec(memory_space=pltpu.SMEM),
    ],
    out_shape=[jax.ShapeDtypeStruct((1, 1), jnp.float32)] * 2,
    scratch_shapes=(
        [pltpu.VMEM((4 * D, 8 * D), jnp.float32)] * 4
        + [pltpu.VMEM((8 * D, 4 * D), jnp.float32)] * 4
    ),
)


def kernel(ausers, aitems, aratings, tusers, titems, tratings, params):
    p = params
    au2 = ausers.reshape(_IDX_ROWS, _CH)
    tu2 = tusers.reshape(_IDX_ROWS, _CH)
    ai2 = aitems.reshape(_IDX_ROWS, _CH)
    ti2 = titems.reshape(_IDX_ROWS, _CH)
    (aue_g,) = _g1(au2, p["a_emb_user"])
    (tue_g,) = _g1(tu2, p["t_emb_user"])
    (aie_g,) = _g1(ai2, p["a_emb_item"])
    (tie_g,) = _g1(ti2, p["t_emb_item"])
    waa_g, wat_g = _g2(au2, tu2, p["W_a"])
    wba_g, wbt_g = _g2(au2, tu2, p["W_b"])
    gathered = [aue_g, tue_g, aie_g, tie_g, waa_g, wat_g, wba_g, wbt_g]
    packed = [g.reshape(_PK, 4 * D) for g in gathered]

    wargs = []
    for name in ("mlp_a_users", "mlp_t_users", "mlp_a_items", "mlp_t_items"):
        m = p[name]
        wargs += [
            m["W1"],
            m["b1"].reshape(1, 2 * D),
            m["W2"],
            m["b2"].reshape(1, D),
        ]
    sel = (jnp.arange(4 * D)[:, None] // D ==
           jnp.arange(4)[None, :]).astype(jnp.float32)

    ar2 = aratings.reshape(_PK, 4)
    tr2 = tratings.reshape(_PK, 4)
    la, lt = _tc_dense(ar2, tr2, *packed, *wargs, sel)
    return (la[0, 0], lt[0, 0])
